# baseline (speedup 1.0000x reference)
import functools

import jax
import jax.numpy as jnp
from jax import lax
from jax.experimental import pallas as pl
from jax.experimental.pallas import tpu as pltpu
from jax.experimental.pallas import tpu_sc as plsc

_B = 16
_D = 1024
_NC = 2
_NS = 16
_NW = _NC * _NS
_CH = 32
_G = _D // 16
_SC_ROWS = 16384
_TC_R = 512


def _make_segsum(tot, sc_rows):
    rpw = sc_rows // _NW
    nchunk = rpw // _CH
    mesh = plsc.VectorSubcoreMesh(core_axis_name="c", subcore_axis_name="s")

    @functools.partial(
        pl.kernel,
        out_type=jax.ShapeDtypeStruct((_NW * _B, _D), jnp.float32),
        mesh=mesh,
        scratch_types=[
            pltpu.VMEM((2, _CH, _D), jnp.float32),
            pltpu.VMEM((_B, _D), jnp.float32),
            pltpu.VMEM((24,), jnp.int32),
            pltpu.SemaphoreType.DMA,
            pltpu.SemaphoreType.DMA,
        ],
    )
    def seg_kernel(flat_hbm, cu_hbm, parts_hbm, buf, acc, cu_v, sem0, sem1):
        cid = lax.axis_index("c")
        sid = lax.axis_index("s")
        wid = cid * _NS + sid
        base = wid * rpw

        pltpu.sync_copy(cu_hbm, cu_v.at[pl.ds(0, _B + 1)])
        v0 = cu_v[pl.ds(0, 16)]
        v1 = cu_v[pl.ds(8, 16)]
        cu_sc = [v0[b] for b in range(16)] + [v1[8]]

        def _zcol(k, _):
            z = jnp.zeros((16,), jnp.float32)
            for r in range(_B):
                acc[r, pl.ds(k * 16, 16)] = z
            return 0
        lax.fori_loop(0, _G, _zcol, 0)

        sems = (sem0, sem1)
        pltpu.async_copy(flat_hbm.at[pl.ds(base, _CH)], buf.at[0], sem0)
        pltpu.async_copy(flat_hbm.at[pl.ds(base + _CH, _CH)], buf.at[1], sem1)

        def chunk_pair(i2, _):
            for u in range(2):
                i = i2 + u
                c0 = base + i * _CH
                pltpu.make_async_copy(
                    flat_hbm.at[pl.ds(0, _CH)], buf.at[u], sems[u]).wait()
                bufu = buf.at[u]

                s0 = jnp.int32(0)
                for b in range(1, _B + 1):
                    s0 = s0 + jnp.clip(c0 - cu_sc[b] + 1, 0, 1)
                s0 = jnp.clip(s0, 0, _B - 1)
                nxt0 = jnp.int32(tot)
                for b in range(1, _B + 1):
                    nxt0 = jnp.where(s0 + 1 == b, cu_sc[b], nxt0)
                hi0 = jnp.minimum(nxt0 - c0, _CH)

                @pl.when(hi0 >= _CH)
                def _():
                    def d_body(d, _):
                        col = d * 16
                        vs = [bufu[r, pl.ds(col, 16)] for r in range(_CH)]
                        while len(vs) > 1:
                            vs = [vs[j] + vs[j + 1]
                                  for j in range(0, len(vs) - 1, 2)] + (
                                      [vs[-1]] if len(vs) % 2 else [])
                        plsc.addupdate(acc.at[s0, pl.ds(col, 16)], vs[0])
                        return 0

                    lax.fori_loop(0, _G, d_body, 0)

                @pl.when(hi0 < _CH)
                def _():
                    def piece_body(kk, p):
                        x = c0 + p
                        s = jnp.int32(0)
                        for b in range(1, _B + 1):
                            s = s + jnp.clip(x - cu_sc[b] + 1, 0, 1)
                        s = jnp.clip(s, 0, _B - 1)
                        nxt = jnp.int32(tot)
                        for b in range(1, _B + 1):
                            nxt = jnp.where(s + 1 == b, cu_sc[b], nxt)
                        e = jnp.minimum(nxt - c0, _CH)

                        def row_body(r, _):
                            for d in range(_G):
                                v = bufu[r, pl.ds(d * 16, 16)]
                                plsc.addupdate(
                                    acc.at[s, pl.ds(d * 16, 16)], v)
                            return 0

                        lax.fori_loop(p, e, row_body, 0)
                        return jnp.maximum(e, p)

                    lax.fori_loop(0, _B, piece_body, jnp.int32(0))

                @pl.when(i + 2 < nchunk)
                def _():
                    pltpu.async_copy(
                        flat_hbm.at[pl.ds(c0 + 2 * _CH, _CH)], buf.at[u],
                        sems[u])

            return 0

        lax.fori_loop(0, nchunk // 2, lambda k, c: chunk_pair(2 * k, c), 0)
        pltpu.sync_copy(acc, parts_hbm.at[pl.ds(wid * _B, _B)])

    return seg_kernel


def _make_tc_segsum(row0):
    def tc_body(cu_ref, x_ref, o_ref):
        i = pl.program_id(0)
        idx = row0 + i * _TC_R + lax.broadcasted_iota(
            jnp.int32, (_B, _TC_R), 1)
        lo = jnp.concatenate(
            [cu_ref[b].reshape(1, 1) for b in range(_B)], axis=0)
        hi = jnp.concatenate(
            [cu_ref[b + 1].reshape(1, 1) for b in range(_B)], axis=0)
        m = ((idx >= lo) & (idx < hi)).astype(jnp.float32)
        part = jnp.dot(m, x_ref[...], preferred_element_type=jnp.float32)

        @pl.when(i == 0)
        def _():
            o_ref[...] = part

        @pl.when(i > 0)
        def _():
            o_ref[...] += part

    return tc_body


def _tc_segsum(flat, cu):
    nt = (flat.shape[0] - _SC_ROWS) // _TC_R
    blk0 = _SC_ROWS // _TC_R
    return pl.pallas_call(
        _make_tc_segsum(_SC_ROWS),
        grid=(nt,),
        out_shape=jax.ShapeDtypeStruct((_B, _D), jnp.float32),
        in_specs=[
            pl.BlockSpec(memory_space=pltpu.SMEM),
            pl.BlockSpec((_TC_R, _D), lambda i: (i + blk0, 0)),
        ],
        out_specs=pl.BlockSpec((_B, _D), lambda i: (0, 0)),
    )(cu, flat)


def _finish_body(cu_ref, parts_ref, tcp_ref, g_ref, beta_ref, W1_ref, b1_ref,
                 W2_ref, b2_ref, out_ref):
    psum = tcp_ref[...]
    for w in range(_NW):
        psum = psum + parts_ref[pl.ds(w * _B, _B), :]
    row = lax.broadcasted_iota(jnp.int32, (_B, 1), 0)
    lens = jnp.zeros((_B, 1), jnp.float32)
    for b in range(_B):
        lb = (cu_ref[b + 1] - cu_ref[b]).astype(jnp.float32)
        lens = lens + jnp.where(row == b, lb, 0.0)
    pooled = psum / jnp.clip(lens, 1e-6, None)
    mu = jnp.mean(pooled, axis=1, keepdims=True)
    var = jnp.mean((pooled - mu) ** 2, axis=1, keepdims=True)
    normed = (pooled - mu) / jnp.sqrt(var + 1e-5) * g_ref[...] + beta_ref[...]
    h = jnp.maximum(
        jnp.dot(normed, W1_ref[...], preferred_element_type=jnp.float32)
        + b1_ref[...], 0.0)
    out_ref[...] = (
        lax.dot_general(h, W2_ref[...], (((1,), (1,)), ((), ())),
                        preferred_element_type=jnp.float32)
        + b2_ref[...])


def _finish_tc(cu, parts, tcp, g2, beta2, W1, b12, W2t, b22):
    c = W2t.shape[0]
    vspec = pl.BlockSpec(memory_space=pltpu.VMEM)
    return pl.pallas_call(
        _finish_body,
        out_shape=jax.ShapeDtypeStruct((_B, c), jnp.float32),
        in_specs=[pl.BlockSpec(memory_space=pltpu.SMEM)] + [vspec] * 8,
    )(cu, parts, tcp, g2, beta2, W1, b12, W2t, b22)


def kernel(flat, cu_seqlens, ln_gamma, ln_beta, W1, b1, W2, b2):
    cu = cu_seqlens.astype(jnp.int32)
    parts = _make_segsum(flat.shape[0], _SC_ROWS)(flat, cu)
    tcp = _tc_segsum(flat, cu)
    return _finish_tc(
        cu, parts, tcp,
        ln_gamma.reshape(1, -1), ln_beta.reshape(1, -1),
        W1, b1.reshape(1, -1), W2.T, b2.reshape(1, -1))


# speedup vs baseline: 1.0413x; 1.0413x over previous
import functools

import jax
import jax.numpy as jnp
from jax import lax
from jax.experimental import pallas as pl
from jax.experimental.pallas import tpu as pltpu
from jax.experimental.pallas import tpu_sc as plsc

_B = 16
_D = 1024
_NC = 2
_NS = 16
_NW = _NC * _NS
_CH = 32
_G = _D // 16
_SC_ROWS = 12288
_TC_R = 512


def _make_segsum(tot, sc_rows):
    rpw = sc_rows // _NW
    nchunk = rpw // _CH
    mesh = plsc.VectorSubcoreMesh(core_axis_name="c", subcore_axis_name="s")

    @functools.partial(
        pl.kernel,
        out_type=jax.ShapeDtypeStruct((_NW * _B, _D), jnp.float32),
        mesh=mesh,
        scratch_types=[
            pltpu.VMEM((2, _CH, _D), jnp.float32),
            pltpu.VMEM((_B, _D), jnp.float32),
            pltpu.VMEM((24,), jnp.int32),
            pltpu.SemaphoreType.DMA,
            pltpu.SemaphoreType.DMA,
        ],
    )
    def seg_kernel(flat_hbm, cu_hbm, parts_hbm, buf, acc, cu_v, sem0, sem1):
        cid = lax.axis_index("c")
        sid = lax.axis_index("s")
        wid = cid * _NS + sid
        base = wid * rpw

        pltpu.sync_copy(cu_hbm, cu_v.at[pl.ds(0, _B + 1)])
        v0 = cu_v[pl.ds(0, 16)]
        v1 = cu_v[pl.ds(8, 16)]
        cu_sc = [v0[b] for b in range(16)] + [v1[8]]

        def _zcol(k, _):
            z = jnp.zeros((16,), jnp.float32)
            for r in range(_B):
                acc[r, pl.ds(k * 16, 16)] = z
            return 0
        lax.fori_loop(0, _G, _zcol, 0)

        sems = (sem0, sem1)
        pltpu.async_copy(flat_hbm.at[pl.ds(base, _CH)], buf.at[0], sem0)
        pltpu.async_copy(flat_hbm.at[pl.ds(base + _CH, _CH)], buf.at[1], sem1)

        def chunk_pair(i2, _):
            for u in range(2):
                i = i2 + u
                c0 = base + i * _CH
                pltpu.make_async_copy(
                    flat_hbm.at[pl.ds(0, _CH)], buf.at[u], sems[u]).wait()
                bufu = buf.at[u]

                s0 = jnp.int32(0)
                for b in range(1, _B + 1):
                    s0 = s0 + jnp.clip(c0 - cu_sc[b] + 1, 0, 1)
                s0 = jnp.clip(s0, 0, _B - 1)
                nxt0 = jnp.int32(tot)
                for b in range(1, _B + 1):
                    nxt0 = jnp.where(s0 + 1 == b, cu_sc[b], nxt0)
                hi0 = jnp.minimum(nxt0 - c0, _CH)

                @pl.when(hi0 >= _CH)
                def _():
                    def d_body(d, _):
                        col = d * 16
                        vs = [bufu[r, pl.ds(col, 16)] for r in range(_CH)]
                        while len(vs) > 1:
                            vs = [vs[j] + vs[j + 1]
                                  for j in range(0, len(vs) - 1, 2)] + (
                                      [vs[-1]] if len(vs) % 2 else [])
                        plsc.addupdate(acc.at[s0, pl.ds(col, 16)], vs[0])
                        return 0

                    lax.fori_loop(0, _G, d_body, 0)

                @pl.when(hi0 < _CH)
                def _():
                    def piece_body(kk, p):
                        x = c0 + p
                        s = jnp.int32(0)
                        for b in range(1, _B + 1):
                            s = s + jnp.clip(x - cu_sc[b] + 1, 0, 1)
                        s = jnp.clip(s, 0, _B - 1)
                        nxt = jnp.int32(tot)
                        for b in range(1, _B + 1):
                            nxt = jnp.where(s + 1 == b, cu_sc[b], nxt)
                        e = jnp.minimum(nxt - c0, _CH)

                        def row_body(r, _):
                            for d in range(_G):
                                v = bufu[r, pl.ds(d * 16, 16)]
                                plsc.addupdate(
                                    acc.at[s, pl.ds(d * 16, 16)], v)
                            return 0

                        lax.fori_loop(p, e, row_body, 0)
                        return jnp.maximum(e, p)

                    lax.fori_loop(0, _B, piece_body, jnp.int32(0))

                @pl.when(i + 2 < nchunk)
                def _():
                    pltpu.async_copy(
                        flat_hbm.at[pl.ds(c0 + 2 * _CH, _CH)], buf.at[u],
                        sems[u])

            return 0

        lax.fori_loop(0, nchunk // 2, lambda k, c: chunk_pair(2 * k, c), 0)
        pltpu.sync_copy(acc, parts_hbm.at[pl.ds(wid * _B, _B)])

    return seg_kernel


def _make_tc_segsum(row0):
    def tc_body(cu_ref, x_ref, o_ref):
        i = pl.program_id(0)
        idx = row0 + i * _TC_R + lax.broadcasted_iota(
            jnp.int32, (_B, _TC_R), 1)
        lo = jnp.concatenate(
            [cu_ref[b].reshape(1, 1) for b in range(_B)], axis=0)
        hi = jnp.concatenate(
            [cu_ref[b + 1].reshape(1, 1) for b in range(_B)], axis=0)
        m = ((idx >= lo) & (idx < hi)).astype(jnp.float32)
        part = jnp.dot(m, x_ref[...], preferred_element_type=jnp.float32)

        @pl.when(i == 0)
        def _():
            o_ref[...] = part

        @pl.when(i > 0)
        def _():
            o_ref[...] += part

    return tc_body


def _tc_segsum(flat, cu):
    nt = (flat.shape[0] - _SC_ROWS) // _TC_R
    blk0 = _SC_ROWS // _TC_R
    return pl.pallas_call(
        _make_tc_segsum(_SC_ROWS),
        grid=(nt,),
        out_shape=jax.ShapeDtypeStruct((_B, _D), jnp.float32),
        in_specs=[
            pl.BlockSpec(memory_space=pltpu.SMEM),
            pl.BlockSpec((_TC_R, _D), lambda i: (i + blk0, 0)),
        ],
        out_specs=pl.BlockSpec((_B, _D), lambda i: (0, 0)),
    )(cu, flat)


def _finish_body(cu_ref, parts_ref, tcp_ref, g_ref, beta_ref, W1_ref, b1_ref,
                 W2_ref, b2_ref, out_ref):
    psum = tcp_ref[...]
    for w in range(_NW):
        psum = psum + parts_ref[pl.ds(w * _B, _B), :]
    row = lax.broadcasted_iota(jnp.int32, (_B, 1), 0)
    lens = jnp.zeros((_B, 1), jnp.float32)
    for b in range(_B):
        lb = (cu_ref[b + 1] - cu_ref[b]).astype(jnp.float32)
        lens = lens + jnp.where(row == b, lb, 0.0)
    pooled = psum / jnp.clip(lens, 1e-6, None)
    mu = jnp.mean(pooled, axis=1, keepdims=True)
    var = jnp.mean((pooled - mu) ** 2, axis=1, keepdims=True)
    normed = (pooled - mu) / jnp.sqrt(var + 1e-5) * g_ref[...] + beta_ref[...]
    h = jnp.maximum(
        jnp.dot(normed, W1_ref[...], preferred_element_type=jnp.float32)
        + b1_ref[...], 0.0)
    out_ref[...] = (
        lax.dot_general(h, W2_ref[...], (((1,), (1,)), ((), ())),
                        preferred_element_type=jnp.float32)
        + b2_ref[...])


def _finish_tc(cu, parts, tcp, g2, beta2, W1, b12, W2t, b22):
    c = W2t.shape[0]
    vspec = pl.BlockSpec(memory_space=pltpu.VMEM)
    return pl.pallas_call(
        _finish_body,
        out_shape=jax.ShapeDtypeStruct((_B, c), jnp.float32),
        in_specs=[pl.BlockSpec(memory_space=pltpu.SMEM)] + [vspec] * 8,
    )(cu, parts, tcp, g2, beta2, W1, b12, W2t, b22)


def kernel(flat, cu_seqlens, ln_gamma, ln_beta, W1, b1, W2, b2):
    cu = cu_seqlens.astype(jnp.int32)
    parts = _make_segsum(flat.shape[0], _SC_ROWS)(flat, cu)
    tcp = _tc_segsum(flat, cu)
    return _finish_tc(
        cu, parts, tcp,
        ln_gamma.reshape(1, -1), ln_beta.reshape(1, -1),
        W1, b1.reshape(1, -1), W2.T, b2.reshape(1, -1))


# speedup vs baseline: 1.0768x; 1.0341x over previous
import functools

import jax
import jax.numpy as jnp
from jax import lax
from jax.experimental import pallas as pl
from jax.experimental.pallas import tpu as pltpu
from jax.experimental.pallas import tpu_sc as plsc

_B = 16
_D = 1024
_NC = 2
_NS = 16
_NW = _NC * _NS
_CH = 32
_G = _D // 16
_SC_ROWS = 12288
_TC_R = 1024


def _make_segsum(tot, sc_rows):
    rpw = sc_rows // _NW
    nchunk = rpw // _CH
    mesh = plsc.VectorSubcoreMesh(core_axis_name="c", subcore_axis_name="s")

    @functools.partial(
        pl.kernel,
        out_type=jax.ShapeDtypeStruct((_NW * _B, _D), jnp.float32),
        mesh=mesh,
        scratch_types=[
            pltpu.VMEM((2, _CH, _D), jnp.float32),
            pltpu.VMEM((_B, _D), jnp.float32),
            pltpu.VMEM((24,), jnp.int32),
            pltpu.SemaphoreType.DMA,
            pltpu.SemaphoreType.DMA,
        ],
    )
    def seg_kernel(flat_hbm, cu_hbm, parts_hbm, buf, acc, cu_v, sem0, sem1):
        cid = lax.axis_index("c")
        sid = lax.axis_index("s")
        wid = cid * _NS + sid
        base = wid * rpw

        pltpu.sync_copy(cu_hbm, cu_v.at[pl.ds(0, _B + 1)])
        v0 = cu_v[pl.ds(0, 16)]
        v1 = cu_v[pl.ds(8, 16)]
        cu_sc = [v0[b] for b in range(16)] + [v1[8]]

        def _zcol(k, _):
            z = jnp.zeros((16,), jnp.float32)
            for r in range(_B):
                acc[r, pl.ds(k * 16, 16)] = z
            return 0
        lax.fori_loop(0, _G, _zcol, 0)

        sems = (sem0, sem1)
        pltpu.async_copy(flat_hbm.at[pl.ds(base, _CH)], buf.at[0], sem0)
        pltpu.async_copy(flat_hbm.at[pl.ds(base + _CH, _CH)], buf.at[1], sem1)

        def chunk_pair(i2, _):
            for u in range(2):
                i = i2 + u
                c0 = base + i * _CH
                pltpu.make_async_copy(
                    flat_hbm.at[pl.ds(0, _CH)], buf.at[u], sems[u]).wait()
                bufu = buf.at[u]

                s0 = jnp.int32(0)
                for b in range(1, _B + 1):
                    s0 = s0 + jnp.clip(c0 - cu_sc[b] + 1, 0, 1)
                s0 = jnp.clip(s0, 0, _B - 1)
                nxt0 = jnp.int32(tot)
                for b in range(1, _B + 1):
                    nxt0 = jnp.where(s0 + 1 == b, cu_sc[b], nxt0)
                hi0 = jnp.minimum(nxt0 - c0, _CH)

                @pl.when(hi0 >= _CH)
                def _():
                    def d_body(d, _):
                        col = d * 16
                        vs = [bufu[r, pl.ds(col, 16)] for r in range(_CH)]
                        while len(vs) > 1:
                            vs = [vs[j] + vs[j + 1]
                                  for j in range(0, len(vs) - 1, 2)] + (
                                      [vs[-1]] if len(vs) % 2 else [])
                        plsc.addupdate(acc.at[s0, pl.ds(col, 16)], vs[0])
                        return 0

                    lax.fori_loop(0, _G, d_body, 0)

                @pl.when(hi0 < _CH)
                def _():
                    def piece_body(kk, p):
                        x = c0 + p
                        s = jnp.int32(0)
                        for b in range(1, _B + 1):
                            s = s + jnp.clip(x - cu_sc[b] + 1, 0, 1)
                        s = jnp.clip(s, 0, _B - 1)
                        nxt = jnp.int32(tot)
                        for b in range(1, _B + 1):
                            nxt = jnp.where(s + 1 == b, cu_sc[b], nxt)
                        e = jnp.minimum(nxt - c0, _CH)

                        def row_body(r, _):
                            for d in range(_G):
                                v = bufu[r, pl.ds(d * 16, 16)]
                                plsc.addupdate(
                                    acc.at[s, pl.ds(d * 16, 16)], v)
                            return 0

                        lax.fori_loop(p, e, row_body, 0)
                        return jnp.maximum(e, p)

                    lax.fori_loop(0, _B, piece_body, jnp.int32(0))

                @pl.when(i + 2 < nchunk)
                def _():
                    pltpu.async_copy(
                        flat_hbm.at[pl.ds(c0 + 2 * _CH, _CH)], buf.at[u],
                        sems[u])

            return 0

        lax.fori_loop(0, nchunk // 2, lambda k, c: chunk_pair(2 * k, c), 0)
        pltpu.sync_copy(acc, parts_hbm.at[pl.ds(wid * _B, _B)])

    return seg_kernel


def _make_tc_segsum(row0):
    def tc_body(cu_ref, x_ref, o_ref):
        i = pl.program_id(0)
        idx = row0 + i * _TC_R + lax.broadcasted_iota(
            jnp.int32, (_B, _TC_R), 1)
        lo = jnp.concatenate(
            [cu_ref[b].reshape(1, 1) for b in range(_B)], axis=0)
        hi = jnp.concatenate(
            [cu_ref[b + 1].reshape(1, 1) for b in range(_B)], axis=0)
        m = ((idx >= lo) & (idx < hi)).astype(jnp.float32)
        part = jnp.dot(m, x_ref[...], preferred_element_type=jnp.float32)

        @pl.when(i == 0)
        def _():
            o_ref[...] = part

        @pl.when(i > 0)
        def _():
            o_ref[...] += part

    return tc_body


def _tc_segsum(flat, cu):
    nt = (flat.shape[0] - _SC_ROWS) // _TC_R
    blk0 = _SC_ROWS // _TC_R
    return pl.pallas_call(
        _make_tc_segsum(_SC_ROWS),
        grid=(nt,),
        out_shape=jax.ShapeDtypeStruct((_B, _D), jnp.float32),
        in_specs=[
            pl.BlockSpec(memory_space=pltpu.SMEM),
            pl.BlockSpec((_TC_R, _D), lambda i: (i + blk0, 0)),
        ],
        out_specs=pl.BlockSpec((_B, _D), lambda i: (0, 0)),
    )(cu, flat)


def _finish_body(cu_ref, parts_ref, tcp_ref, g_ref, beta_ref, W1_ref, b1_ref,
                 W2_ref, b2_ref, out_ref):
    psum = tcp_ref[...]
    for w in range(_NW):
        psum = psum + parts_ref[pl.ds(w * _B, _B), :]
    row = lax.broadcasted_iota(jnp.int32, (_B, 1), 0)
    lens = jnp.zeros((_B, 1), jnp.float32)
    for b in range(_B):
        lb = (cu_ref[b + 1] - cu_ref[b]).astype(jnp.float32)
        lens = lens + jnp.where(row == b, lb, 0.0)
    pooled = psum / jnp.clip(lens, 1e-6, None)
    mu = jnp.mean(pooled, axis=1, keepdims=True)
    var = jnp.mean((pooled - mu) ** 2, axis=1, keepdims=True)
    normed = (pooled - mu) / jnp.sqrt(var + 1e-5) * g_ref[...] + beta_ref[...]
    h = jnp.maximum(
        jnp.dot(normed, W1_ref[...], preferred_element_type=jnp.float32)
        + b1_ref[...], 0.0)
    out_ref[...] = (
        lax.dot_general(h, W2_ref[...], (((1,), (1,)), ((), ())),
                        preferred_element_type=jnp.float32)
        + b2_ref[...])


def _finish_tc(cu, parts, tcp, g2, beta2, W1, b12, W2t, b22):
    c = W2t.shape[0]
    vspec = pl.BlockSpec(memory_space=pltpu.VMEM)
    return pl.pallas_call(
        _finish_body,
        out_shape=jax.ShapeDtypeStruct((_B, c), jnp.float32),
        in_specs=[pl.BlockSpec(memory_space=pltpu.SMEM)] + [vspec] * 8,
    )(cu, parts, tcp, g2, beta2, W1, b12, W2t, b22)


def kernel(flat, cu_seqlens, ln_gamma, ln_beta, W1, b1, W2, b2):
    cu = cu_seqlens.astype(jnp.int32)
    parts = _make_segsum(flat.shape[0], _SC_ROWS)(flat, cu)
    tcp = _tc_segsum(flat, cu)
    return _finish_tc(
        cu, parts, tcp,
        ln_gamma.reshape(1, -1), ln_beta.reshape(1, -1),
        W1, b1.reshape(1, -1), W2.T, b2.reshape(1, -1))
